# baseline (speedup 1.0000x reference)
import functools

import jax
import jax.numpy as jnp
from jax import lax
from jax.experimental import pallas as pl
from jax.experimental.pallas import tpu as pltpu
from jax.experimental.pallas import tpu_sc as plsc

N = 10000
R = 6
E = 25000
EP = 25600
HID = 512
L = 16
NT = 32
NPT = 320
NPAD = NT * NPT
CAP = 2048
CH = 1600
NCH = EP // CH
KCH = 32
FH = 256
PAD_DST = 2 ** 30


def _tc_common(hv, w_ref, b_ref, dw_ref, db_ref, wh_ref, self_ref):
    acc = jnp.dot(hv, w_ref[...], preferred_element_type=jnp.float32)
    acc = acc + b_ref[...]
    self_ref[...] = acc[:, 2 * HID:]
    v0 = acc[:, :HID]
    v1 = acc[:, HID:2 * HID]
    dw = dw_ref[...]
    db = db_ref[...]
    for r in range(R):
        wh_ref[r] = dw[r, 0] * v0 + dw[r, 1] * v1 + db[0, r]


def _tc_l1_kernel(h_ref, w_ref, b_ref, dw_ref, db_ref, wh_ref, self_ref):
    _tc_common(h_ref[...], w_ref, b_ref, dw_ref, db_ref, wh_ref, self_ref)


def _tc_l2_kernel(agg_ref, sin_ref, w_ref, b_ref, dw_ref, db_ref, wh_ref,
                  self_ref):
    t = agg_ref[...] + sin_ref[...]
    hv = jnp.where(t >= 0, t, 0.01 * t)
    _tc_common(hv, w_ref, b_ref, dw_ref, db_ref, wh_ref, self_ref)


def _tc_final_kernel(agg_ref, sin_ref, o_ref):
    t = agg_ref[...] + sin_ref[...]
    o_ref[...] = jnp.where(t >= 0, t, 0.01 * t)


def _tc_layer(inputs, kernel_fn, din, baseW, baseB, dec_w, dec_b, statW,
              statB):
    wcat = jnp.concatenate([baseW[0], baseW[1], statW], axis=1)
    bcat = jnp.concatenate([baseB[0], baseB[1], statB])[None, :]
    dbr = dec_b[None, :]
    bn = 1000
    n_in = len(inputs)
    in_specs = [pl.BlockSpec((bn, din), lambda i: (i, 0))] * n_in + [
        pl.BlockSpec((din, 3 * HID), lambda i: (0, 0)),
        pl.BlockSpec((1, 3 * HID), lambda i: (0, 0)),
        pl.BlockSpec((R, 2), lambda i: (0, 0)),
        pl.BlockSpec((1, R), lambda i: (0, 0)),
    ]
    wh, slf = pl.pallas_call(
        kernel_fn,
        grid=(N // bn,),
        in_specs=in_specs,
        out_specs=[pl.BlockSpec((R, bn, HID), lambda i: (0, i, 0)),
                   pl.BlockSpec((bn, HID), lambda i: (i, 0))],
        out_shape=[jax.ShapeDtypeStruct((R, N, HID), jnp.float32),
                   jax.ShapeDtypeStruct((N, HID), jnp.float32)],
    )(*inputs, wcat, bcat, dec_w, dbr)
    return wh, slf


def _tc_final(agg, slf):
    bn = 1000
    return pl.pallas_call(
        _tc_final_kernel,
        grid=(N // bn,),
        in_specs=[pl.BlockSpec((bn, HID), lambda i: (i, 0)),
                  pl.BlockSpec((bn, HID), lambda i: (i, 0))],
        out_specs=pl.BlockSpec((bn, HID), lambda i: (i, 0)),
        out_shape=jax.ShapeDtypeStruct((N, HID), jnp.float32),
    )(agg, slf)


def _sc_body(table_h, src_h, dst_h, out_h, acc, hxl, dll, sstg, dstg, cnt,
             gidx, rowbuf, sem):
    cc = lax.axis_index("c")
    ss = lax.axis_index("s")
    wid = cc * 16 + ss
    lo = wid * NPT

    zi = jnp.zeros((L,), jnp.int32)
    zf = jnp.zeros((L,), jnp.float32)
    ones = jnp.ones((L,), jnp.float32)
    dummy = jnp.full((L,), NPT, jnp.int32)
    lane = lax.iota(jnp.int32, L)

    def mz(i, carry):
        hxl[pl.ds(i * L, L)] = zi
        dll[pl.ds(i * L, L)] = dummy
        return carry
    lax.fori_loop(0, R * CAP // L + 1, mz, 0)

    totals = []
    for r in range(R):
        def chunk_body(ci, pv, r=r):
            off = pl.multiple_of(r * EP + ci * CH, 8)
            pltpu.sync_copy(src_h.at[pl.ds(off, CH)], sstg)
            pltpu.sync_copy(dst_h.at[pl.ds(off, CH)], dstg)

            def g_body(g, pv):
                dv = dstg[pl.ds(g * L, L)]
                sv = sstg[pl.ds(g * L, L)]
                m = (dv >= lo) & (dv < lo + NPT) & (pv < CAP // L)
                hv = 2 * (r * N) + 2 * sv
                dlv = dv - lo
                idx = r * CAP + pv * L + lane
                plsc.store_scatter(hxl, [idx], hv, mask=m)
                plsc.store_scatter(dll, [idx], dlv, mask=m)
                return pv + m.astype(jnp.int32)
            return lax.fori_loop(0, CH // L, g_body, pv)
        pv = lax.fori_loop(0, NCH, chunk_body, zi)
        tot = pv[0]
        for j in range(1, L):
            tot = jnp.maximum(tot, pv[j])
        totals.append(tot * L)

    for c in (0, 1):
        def za(i, carry):
            acc[pl.ds(i * L, L)] = zf
            return carry
        lax.fori_loop(0, (NPT + 1) * FH // L, za, 0)

        for r in range(R):
            tot = totals[r]

            def zc(i, carry):
                cnt[pl.ds(i * L, L)] = zf
                return carry
            lax.fori_loop(0, NPT + 1, zc, 0)

            def cb(k, carry, r=r):
                d = dll[pl.ds(r * CAP + k, L)][0]
                plsc.addupdate(cnt.at[pl.ds(d * L, L)], ones)
                return carry
            lax.fori_loop(0, tot, cb, 0)

            def rb(i, carry):
                v = cnt[pl.ds(i * L, L)]
                cnt[pl.ds(i * L, L)] = 1.0 / jnp.maximum(v, 1.0)
                return carry
            lax.fori_loop(0, NPT + 1, rb, 0)

            nq = (tot + KCH - 1) // KCH

            def qb(q, carry, r=r, c=c):
                base = q * KCH
                for j in range(KCH // L):
                    gidx[pl.ds(j * L, L)] = (
                        hxl[pl.ds(r * CAP + base + j * L, L)] + c)
                pltpu.async_copy(table_h.at[gidx], rowbuf, sem).wait()

                def eb(k, carry, r=r):
                    d = dll[pl.ds(r * CAP + base + k, L)][0]
                    w = cnt[pl.ds(d * L, L)][0]
                    for j in range(FH // L):
                        plsc.addupdate(
                            acc.at[pl.ds(d * FH + j * L, L)],
                            rowbuf[k, pl.ds(j * L, L)] * w)
                    return carry
                lax.fori_loop(0, KCH, eb, 0)
                return carry
            lax.fori_loop(0, nq, qb, 0)

        off = pl.multiple_of(c * NPAD * FH + lo * FH, 8)
        pltpu.sync_copy(acc.at[pl.ds(0, NPT * FH)],
                        out_h.at[pl.ds(off, NPT * FH)])


def _sc_agg(table, srcp, dstp):
    mesh = plsc.VectorSubcoreMesh(core_axis_name="c", subcore_axis_name="s")
    k = functools.partial(
        pl.kernel,
        out_type=jax.ShapeDtypeStruct((2 * NPAD * FH,), jnp.float32),
        mesh=mesh,
        compiler_params=pltpu.CompilerParams(needs_layout_passes=False),
        scratch_types=[
            pltpu.VMEM(((NPT + 1) * FH,), jnp.float32),
            pltpu.VMEM((R * CAP + L,), jnp.int32),
            pltpu.VMEM((R * CAP + L,), jnp.int32),
            pltpu.VMEM((CH,), jnp.int32),
            pltpu.VMEM((CH,), jnp.int32),
            pltpu.VMEM(((NPT + 1) * L,), jnp.float32),
            pltpu.VMEM((KCH,), jnp.int32),
            pltpu.VMEM((KCH, FH), jnp.float32),
            pltpu.SemaphoreType.DMA,
        ],
    )(_sc_body)
    return k(table, srcp, dstp)


def _layer_agg(wh, srcp, dstp):
    table = wh.reshape(R * N * 2, FH)
    out = _sc_agg(table, srcp, dstp).reshape(2, NPAD, FH)
    return jnp.concatenate([out[0, :N], out[1, :N]], axis=1)


def kernel(x, edge_index, l1_baseW, l1_baseB, l1_dec_w, l1_dec_b, l1_statW,
           l1_statB, l2_baseW, l2_baseB, l2_dec_w, l2_dec_b, l2_statW,
           l2_statB):
    ei = edge_index.astype(jnp.int32)
    srcp = jnp.pad(ei[:, 0, :], ((0, 0), (0, EP - E))).reshape(-1)
    dstp = jnp.pad(ei[:, 1, :], ((0, 0), (0, EP - E)),
                   constant_values=PAD_DST).reshape(-1)

    wh1, slf1 = _tc_layer((x,), _tc_l1_kernel, 768, l1_baseW, l1_baseB,
                          l1_dec_w, l1_dec_b, l1_statW, l1_statB)
    agg1 = _layer_agg(wh1, srcp, dstp)
    wh2, slf2 = _tc_layer((agg1, slf1), _tc_l2_kernel, HID, l2_baseW,
                          l2_baseB, l2_dec_w, l2_dec_b, l2_statW, l2_statB)
    agg2 = _layer_agg(wh2, srcp, dstp)
    return _tc_final(agg2, slf2)


# speedup vs baseline: 2.7955x; 2.7955x over previous
import functools

import jax
import jax.numpy as jnp
from jax import lax
from jax.experimental import pallas as pl
from jax.experimental.pallas import tpu as pltpu
from jax.experimental.pallas import tpu_sc as plsc

N = 10000
R = 6
E = 25000
EP = 25600
HID = 512
L = 16
NT = 32
NSUB = 2
NPS = 160
NPT = NSUB * NPS
NPAD = NT * NPT
CAP = 768
CH = 2560
NCH = EP // CH
KCH = 16
PAD_DST = 2 ** 30


def _tc_common(hv, w_ref, b_ref, dw_ref, db_ref, wh_ref, self_ref):
    acc = jnp.dot(hv, w_ref[...], preferred_element_type=jnp.float32)
    acc = acc + b_ref[...]
    self_ref[...] = acc[:, 2 * HID:]
    v0 = acc[:, :HID]
    v1 = acc[:, HID:2 * HID]
    dw = dw_ref[...]
    db = db_ref[...]
    for r in range(R):
        wh_ref[r] = dw[r, 0] * v0 + dw[r, 1] * v1 + db[0, r]


def _tc_l1_kernel(h_ref, w_ref, b_ref, dw_ref, db_ref, wh_ref, self_ref):
    _tc_common(h_ref[...], w_ref, b_ref, dw_ref, db_ref, wh_ref, self_ref)


def _tc_l2_kernel(agg_ref, sin_ref, w_ref, b_ref, dw_ref, db_ref, wh_ref,
                  self_ref):
    t = agg_ref[...] + sin_ref[...]
    hv = jnp.where(t >= 0, t, 0.01 * t)
    _tc_common(hv, w_ref, b_ref, dw_ref, db_ref, wh_ref, self_ref)


def _tc_final_kernel(agg_ref, sin_ref, o_ref):
    t = agg_ref[...] + sin_ref[...]
    o_ref[...] = jnp.where(t >= 0, t, 0.01 * t)


def _tc_layer(inputs, kernel_fn, din, baseW, baseB, dec_w, dec_b, statW,
              statB):
    wcat = jnp.concatenate([baseW[0], baseW[1], statW], axis=1)
    bcat = jnp.concatenate([baseB[0], baseB[1], statB])[None, :]
    dbr = dec_b[None, :]
    bn = 1000
    n_in = len(inputs)
    in_specs = [pl.BlockSpec((bn, din), lambda i: (i, 0))] * n_in + [
        pl.BlockSpec((din, 3 * HID), lambda i: (0, 0)),
        pl.BlockSpec((1, 3 * HID), lambda i: (0, 0)),
        pl.BlockSpec((R, 2), lambda i: (0, 0)),
        pl.BlockSpec((1, R), lambda i: (0, 0)),
    ]
    wh, slf = pl.pallas_call(
        kernel_fn,
        grid=(N // bn,),
        in_specs=in_specs,
        out_specs=[pl.BlockSpec((R, bn, HID), lambda i: (0, i, 0)),
                   pl.BlockSpec((bn, HID), lambda i: (i, 0))],
        out_shape=[jax.ShapeDtypeStruct((R, N, HID), jnp.float32),
                   jax.ShapeDtypeStruct((N, HID), jnp.float32)],
    )(*inputs, wcat, bcat, dec_w, dbr)
    return wh, slf


def _tc_final(agg, slf):
    bn = 1000
    return pl.pallas_call(
        _tc_final_kernel,
        grid=(N // bn,),
        in_specs=[pl.BlockSpec((bn, HID), lambda i: (i, 0)),
                  pl.BlockSpec((bn, HID), lambda i: (i, 0))],
        out_specs=pl.BlockSpec((bn, HID), lambda i: (i, 0)),
        out_shape=jax.ShapeDtypeStruct((N, HID), jnp.float32),
    )(agg, slf)


def _sc_body(table_h, src_h, dst_h, out_h, acc, hxl, dll, sstg, dstg, cnt,
             gidx0, gidx1, rb0, rb1, curs_s, sem0, sem1):
    cc = lax.axis_index("c")
    ss = lax.axis_index("s")
    wid = cc * 16 + ss
    lo = wid * NPT

    zi = jnp.zeros((L,), jnp.int32)
    zf = jnp.zeros((L,), jnp.float32)
    ones = jnp.ones((L,), jnp.float32)

    def mz(i, carry):
        hxl[pl.ds(i * L, L)] = zi
        return carry
    lax.fori_loop(0, (NSUB * R * CAP) // L + 1, mz, 0)

    for r in range(R):
        def chunk_body(ci, cs, r=r):
            off = pl.multiple_of(r * EP + ci * CH, 8)
            pltpu.sync_copy(src_h.at[pl.ds(off, CH)], sstg)
            pltpu.sync_copy(dst_h.at[pl.ds(off, CH)], dstg)

            def g_body(g, cs, r=r):
                c0, c1 = cs
                dv = dstg[pl.ds(g * L, L)]
                sv = sstg[pl.ds(g * L, L)]
                dloc = dv - lo
                hv = r * N + sv
                m0 = (dloc >= 0) & (dloc < NPS) & (c0 <= CAP - L)
                m1 = (dloc >= NPS) & (dloc < 2 * NPS) & (c1 <= CAP - L)
                n0 = jnp.sum(m0.astype(jnp.int32))
                n1 = jnp.sum(m1.astype(jnp.int32))
                s0 = r * CAP
                s1 = (R + r) * CAP
                plsc.store_compressed(hxl.at[pl.ds(s0 + c0, L)], hv, mask=m0)
                plsc.store_compressed(dll.at[pl.ds(s0 + c0, L)], dloc,
                                      mask=m0)
                plsc.store_compressed(hxl.at[pl.ds(s1 + c1, L)], hv, mask=m1)
                plsc.store_compressed(dll.at[pl.ds(s1 + c1, L)], dloc - NPS,
                                      mask=m1)
                return (c0 + n0, c1 + n1)
            return lax.fori_loop(0, CH // L, g_body, cs)
        c0, c1 = lax.fori_loop(0, NCH, chunk_body,
                               (jnp.int32(0), jnp.int32(0)))
        curs_s[r] = c0
        curs_s[R + r] = c1

    def pp(p, carry):
        sub = p // R
        r = p - sub * R
        losub = lo + sub * NPS
        seg = p * CAP
        cur = curs_s[p]

        @pl.when(r == 0)
        def _():
            def za(i, carry):
                acc[pl.ds(i * L, L)] = zf
                return carry
            lax.fori_loop(0, NPS * HID // L, za, 0)

        def zc(i, carry):
            cnt[pl.ds(i * L, L)] = zf
            return carry
        lax.fori_loop(0, NPS, zc, 0)

        def cb(k, carry):
            d = dll[pl.ds(seg + k, L)][0]
            plsc.addupdate(cnt.at[pl.ds(d * L, L)], ones)
            return carry
        lax.fori_loop(0, cur, cb, 0)

        def rb(i, carry):
            v = cnt[pl.ds(i * L, L)]
            cnt[pl.ds(i * L, L)] = 1.0 / jnp.maximum(v, 1.0)
            return carry
        lax.fori_loop(0, NPS, rb, 0)

        nq = (cur + KCH - 1) // KCH
        bufs = ((gidx0, rb0, sem0), (gidx1, rb1, sem1))

        def fire(q, b):
            g, rbuf, sem = bufs[b]
            g[pl.ds(0, L)] = hxl[pl.ds(seg + q * KCH, L)]
            pltpu.async_copy(table_h.at[g], rbuf, sem)

        def drain(b):
            g, rbuf, sem = bufs[b]
            pltpu.make_async_copy(table_h.at[g], rbuf, sem).wait()

        def process(q, b):
            _, rbuf, _ = bufs[b]
            base = q * KCH
            kn = jnp.minimum(cur - base, KCH)

            def eb(k, carry):
                d = dll[pl.ds(seg + base + k, L)][0]
                w = cnt[pl.ds(d * L, L)][0]
                for j in range(HID // L):
                    plsc.addupdate(acc.at[pl.ds(d * HID + j * L, L)],
                                   rbuf[k, pl.ds(j * L, L)] * w)
                return carry
            lax.fori_loop(0, kn, eb, 0)

        @pl.when(nq > 0)
        def _():
            fire(0, 0)

        def pair(qq, carry):
            q0 = 2 * qq

            @pl.when(q0 + 1 < nq)
            def _():
                fire(q0 + 1, 1)
            drain(0)
            process(q0, 0)

            @pl.when(q0 + 2 < nq)
            def _():
                fire(q0 + 2, 0)

            @pl.when(q0 + 1 < nq)
            def _():
                drain(1)
                process(q0 + 1, 1)
            return carry
        lax.fori_loop(0, (nq + 1) // 2, pair, 0)

        @pl.when(r == R - 1)
        def _():
            off = pl.multiple_of(losub * HID, 8)
            pltpu.sync_copy(acc, out_h.at[pl.ds(off, NPS * HID)])
        return carry
    lax.fori_loop(0, NSUB * R, pp, 0)


def _sc_agg(table, srcp, dstp):
    mesh = plsc.VectorSubcoreMesh(core_axis_name="c", subcore_axis_name="s")
    k = functools.partial(
        pl.kernel,
        out_type=jax.ShapeDtypeStruct((NPAD * HID,), jnp.float32),
        mesh=mesh,
        compiler_params=pltpu.CompilerParams(needs_layout_passes=False),
        scratch_types=[
            pltpu.VMEM((NPS * HID,), jnp.float32),
            pltpu.VMEM((NSUB * R * CAP + L,), jnp.int32),
            pltpu.VMEM((NSUB * R * CAP + L,), jnp.int32),
            pltpu.VMEM((CH,), jnp.int32),
            pltpu.VMEM((CH,), jnp.int32),
            pltpu.VMEM((NPS * L,), jnp.float32),
            pltpu.VMEM((KCH,), jnp.int32),
            pltpu.VMEM((KCH,), jnp.int32),
            pltpu.VMEM((KCH, HID), jnp.float32),
            pltpu.VMEM((KCH, HID), jnp.float32),
            pltpu.SMEM((NSUB * R,), jnp.int32),
            pltpu.SemaphoreType.DMA,
            pltpu.SemaphoreType.DMA,
        ],
    )(_sc_body)
    return k(table, srcp, dstp)


def _layer_agg(wh, srcp, dstp):
    table = wh.reshape(R * N, HID)
    return _sc_agg(table, srcp, dstp).reshape(NPAD, HID)[:N]


def kernel(x, edge_index, l1_baseW, l1_baseB, l1_dec_w, l1_dec_b, l1_statW,
           l1_statB, l2_baseW, l2_baseB, l2_dec_w, l2_dec_b, l2_statW,
           l2_statB):
    ei = edge_index.astype(jnp.int32)
    srcp = jnp.pad(ei[:, 0, :], ((0, 0), (0, EP - E))).reshape(-1)
    dstp = jnp.pad(ei[:, 1, :], ((0, 0), (0, EP - E)),
                   constant_values=PAD_DST).reshape(-1)

    wh1, slf1 = _tc_layer((x,), _tc_l1_kernel, 768, l1_baseW, l1_baseB,
                          l1_dec_w, l1_dec_b, l1_statW, l1_statB)
    agg1 = _layer_agg(wh1, srcp, dstp)
    wh2, slf2 = _tc_layer((agg1, slf1), _tc_l2_kernel, HID, l2_baseW,
                          l2_baseB, l2_dec_w, l2_dec_b, l2_statW, l2_statB)
    agg2 = _layer_agg(wh2, srcp, dstp)
    return _tc_final(agg2, slf2)


# speedup vs baseline: 3.4178x; 1.2226x over previous
import functools

import jax
import jax.numpy as jnp
from jax import lax
from jax.experimental import pallas as pl
from jax.experimental.pallas import tpu as pltpu
from jax.experimental.pallas import tpu_sc as plsc

N = 10000
R = 6
E = 25000
EP = 25600
HID = 512
L = 16
NT = 32
NSUB = 2
NPS = 160
NPT = NSUB * NPS
NPAD = NT * NPT
NP_ = NSUB * R
CAP = 576
LSZ = NP_ * CAP
KCH = 16
NBUF = 3
PAD_DST = 2 ** 30


def _tc_common(hv, w_ref, b_ref, dw_ref, db_ref, wh_ref, self_ref):
    acc = jnp.dot(hv, w_ref[...], preferred_element_type=jnp.float32)
    acc = acc + b_ref[...]
    self_ref[...] = acc[:, 2 * HID:]
    v0 = acc[:, :HID]
    v1 = acc[:, HID:2 * HID]
    dw = dw_ref[...]
    db = db_ref[...]
    for r in range(R):
        wh_ref[r] = dw[r, 0] * v0 + dw[r, 1] * v1 + db[0, r]


def _tc_l1_kernel(h_ref, w_ref, b_ref, dw_ref, db_ref, wh_ref, self_ref):
    _tc_common(h_ref[...], w_ref, b_ref, dw_ref, db_ref, wh_ref, self_ref)


def _tc_l2_kernel(agg_ref, sin_ref, w_ref, b_ref, dw_ref, db_ref, wh_ref,
                  self_ref):
    t = agg_ref[...] + sin_ref[...]
    hv = jnp.where(t >= 0, t, 0.01 * t)
    _tc_common(hv, w_ref, b_ref, dw_ref, db_ref, wh_ref, self_ref)


def _tc_final_kernel(agg_ref, sin_ref, o_ref):
    t = agg_ref[...] + sin_ref[...]
    o_ref[...] = jnp.where(t >= 0, t, 0.01 * t)


def _tc_layer(inputs, kernel_fn, din, baseW, baseB, dec_w, dec_b, statW,
              statB):
    wcat = jnp.concatenate([baseW[0], baseW[1], statW], axis=1)
    bcat = jnp.concatenate([baseB[0], baseB[1], statB])[None, :]
    dbr = dec_b[None, :]
    bn = 1000
    n_in = len(inputs)
    in_specs = [pl.BlockSpec((bn, din), lambda i: (i, 0))] * n_in + [
        pl.BlockSpec((din, 3 * HID), lambda i: (0, 0)),
        pl.BlockSpec((1, 3 * HID), lambda i: (0, 0)),
        pl.BlockSpec((R, 2), lambda i: (0, 0)),
        pl.BlockSpec((1, R), lambda i: (0, 0)),
    ]
    wh, slf = pl.pallas_call(
        kernel_fn,
        grid=(N // bn,),
        in_specs=in_specs,
        out_specs=[pl.BlockSpec((R, bn, HID), lambda i: (0, i, 0)),
                   pl.BlockSpec((bn, HID), lambda i: (i, 0))],
        out_shape=[jax.ShapeDtypeStruct((R, N, HID), jnp.float32),
                   jax.ShapeDtypeStruct((N, HID), jnp.float32)],
    )(*inputs, wcat, bcat, dec_w, dbr)
    return wh, slf


def _tc_final(agg, slf):
    bn = 1000
    return pl.pallas_call(
        _tc_final_kernel,
        grid=(N // bn,),
        in_specs=[pl.BlockSpec((bn, HID), lambda i: (i, 0)),
                  pl.BlockSpec((bn, HID), lambda i: (i, 0))],
        out_specs=pl.BlockSpec((bn, HID), lambda i: (i, 0)),
        out_shape=jax.ShapeDtypeStruct((N, HID), jnp.float32),
    )(agg, slf)


def _sc_prep_body(src_h, dst_h, hx_o, dl_o, wl_o, cu_o, hxl, dll, wll, sstg,
                  dstg, cnt, cursv):
    cc = lax.axis_index("c")
    ss = lax.axis_index("s")
    wid = cc * 16 + ss
    lo = wid * NPT

    zi = jnp.zeros((L,), jnp.int32)
    zf = jnp.zeros((L,), jnp.float32)
    ones = jnp.ones((L,), jnp.float32)
    lane = lax.iota(jnp.int32, L)

    def mz(i, carry):
        hxl[pl.ds(i * L, L)] = zi
        dll[pl.ds(i * L, L)] = zi
        wll[pl.ds(i * L, L)] = zf
        return carry
    lax.fori_loop(0, LSZ // L + 1, mz, 0)

    curs = []
    for r in range(R):
        off = pl.multiple_of(r * EP, 8)
        pltpu.sync_copy(src_h.at[pl.ds(off, EP)], sstg)
        pltpu.sync_copy(dst_h.at[pl.ds(off, EP)], dstg)

        def g_body(g, cs, r=r):
            c0, c1 = cs
            dv = dstg[pl.ds(g * L, L)]
            sv = sstg[pl.ds(g * L, L)]
            dloc = dv - lo
            hv = r * N + sv
            m0 = (dloc >= 0) & (dloc < NPS) & (c0 <= CAP - L)
            m1 = (dloc >= NPS) & (dloc < 2 * NPS) & (c1 <= CAP - L)
            n0 = jnp.sum(m0.astype(jnp.int32))
            n1 = jnp.sum(m1.astype(jnp.int32))
            s0 = r * CAP
            s1 = (R + r) * CAP
            plsc.store_compressed(hxl.at[pl.ds(s0 + c0, L)], hv, mask=m0)
            plsc.store_compressed(dll.at[pl.ds(s0 + c0, L)], dloc, mask=m0)
            plsc.store_compressed(hxl.at[pl.ds(s1 + c1, L)], hv, mask=m1)
            plsc.store_compressed(dll.at[pl.ds(s1 + c1, L)], dloc - NPS,
                                  mask=m1)
            return (c0 + n0, c1 + n1)
        c0, c1 = lax.fori_loop(0, EP // L, g_body,
                               (jnp.int32(0), jnp.int32(0)))
        curs.append((c0, c1))

    cv = zi
    for r in range(R):
        cv = jnp.where(lane == r, curs[r][0], cv)
        cv = jnp.where(lane == R + r, curs[r][1], cv)
    cursv[pl.ds(0, L)] = cv
    cursv[pl.ds(L, L)] = zi

    def pp(p, carry):
        seg = p * CAP
        cur = cursv[pl.ds(p, L)][0]

        def zc(i, carry):
            cnt[pl.ds(i * L, L)] = zf
            return carry
        lax.fori_loop(0, NPS, zc, 0)

        def cb(k, carry):
            d = dll[pl.ds(seg + k, L)][0]
            plsc.addupdate(cnt.at[pl.ds(d * L, L)], ones)
            return carry
        lax.fori_loop(0, cur, cb, 0)

        def rb(i, carry):
            v = cnt[pl.ds(i * L, L)]
            cnt[pl.ds(i * L, L)] = 1.0 / jnp.maximum(v, 1.0)
            return carry
        lax.fori_loop(0, NPS, rb, 0)

        def wb(i, carry):
            dv = dll[pl.ds(seg + i * L, L)]
            w16 = plsc.load_gather(cnt, [dv * L])
            wll[pl.ds(seg + i * L, L)] = w16
            return carry
        lax.fori_loop(0, (cur + L - 1) // L, wb, 0)
        return carry
    lax.fori_loop(0, NP_, pp, 0)

    off = pl.multiple_of(wid * LSZ, 8)
    pltpu.sync_copy(hxl.at[pl.ds(0, LSZ)], hx_o.at[pl.ds(off, LSZ)])
    pltpu.sync_copy(dll.at[pl.ds(0, LSZ)], dl_o.at[pl.ds(off, LSZ)])
    pltpu.sync_copy(wll.at[pl.ds(0, LSZ)], wl_o.at[pl.ds(off, LSZ)])
    offc = pl.multiple_of(wid * 2 * L, 8)
    pltpu.sync_copy(cursv, cu_o.at[pl.ds(offc, 2 * L)])


def _sc_prep(srcp, dstp):
    mesh = plsc.VectorSubcoreMesh(core_axis_name="c", subcore_axis_name="s")
    k = functools.partial(
        pl.kernel,
        out_type=[jax.ShapeDtypeStruct((NT * LSZ,), jnp.int32),
                  jax.ShapeDtypeStruct((NT * LSZ,), jnp.int32),
                  jax.ShapeDtypeStruct((NT * LSZ,), jnp.float32),
                  jax.ShapeDtypeStruct((NT * 2 * L,), jnp.int32)],
        mesh=mesh,
        compiler_params=pltpu.CompilerParams(needs_layout_passes=False),
        scratch_types=[
            pltpu.VMEM((LSZ + L,), jnp.int32),
            pltpu.VMEM((LSZ + L,), jnp.int32),
            pltpu.VMEM((LSZ + L,), jnp.float32),
            pltpu.VMEM((EP,), jnp.int32),
            pltpu.VMEM((EP,), jnp.int32),
            pltpu.VMEM((NPS * L,), jnp.float32),
            pltpu.VMEM((2 * L,), jnp.int32),
        ],
    )(_sc_prep_body)
    return k(srcp, dstp)


def _sc_layer_body(table_h, hx_h, dl_h, wl_h, cu_h, out_h, acc, hxl, dll,
                   wll, cursv, g0, g1, g2, rb0, rb1, rb2, sem0, sem1, sem2):
    cc = lax.axis_index("c")
    ss = lax.axis_index("s")
    wid = cc * 16 + ss
    lo = wid * NPT

    zi = jnp.zeros((L,), jnp.int32)
    zf = jnp.zeros((L,), jnp.float32)

    hxl[pl.ds(LSZ, L)] = zi
    off = pl.multiple_of(wid * LSZ, 8)
    pltpu.sync_copy(hx_h.at[pl.ds(off, LSZ)], hxl.at[pl.ds(0, LSZ)])
    pltpu.sync_copy(dl_h.at[pl.ds(off, LSZ)], dll.at[pl.ds(0, LSZ)])
    pltpu.sync_copy(wl_h.at[pl.ds(off, LSZ)], wll.at[pl.ds(0, LSZ)])
    offc = pl.multiple_of(wid * 2 * L, 8)
    pltpu.sync_copy(cu_h.at[pl.ds(offc, 2 * L)], cursv)

    bufs = ((g0, rb0, sem0), (g1, rb1, sem1), (g2, rb2, sem2))

    def pp(p, carry):
        sub = p // R
        r = p - sub * R
        losub = lo + sub * NPS
        seg = p * CAP
        cur = cursv[pl.ds(p, L)][0]

        @pl.when(r == 0)
        def _():
            def za(i, carry):
                acc[pl.ds(i * L, L)] = zf
                return carry
            lax.fori_loop(0, NPS * HID // L, za, 0)

        nq = (cur + KCH - 1) // KCH

        def fire(q, b):
            g, rbuf, sem = bufs[b]
            g[pl.ds(0, L)] = hxl[pl.ds(seg + q * KCH, L)]
            pltpu.async_copy(table_h.at[g], rbuf, sem)

        def drain(b):
            g, rbuf, sem = bufs[b]
            pltpu.make_async_copy(table_h.at[g], rbuf, sem).wait()

        def process(q, b):
            _, rbuf, _ = bufs[b]
            base = q * KCH
            kn = jnp.minimum(cur - base, KCH)

            def eb(k, carry):
                d = dll[pl.ds(seg + base + k, L)][0]
                w = wll[pl.ds(seg + base + k, L)][0]
                for j in range(HID // L):
                    plsc.addupdate(acc.at[pl.ds(d * HID + j * L, L)],
                                   rbuf[k, pl.ds(j * L, L)] * w)
                return carry
            lax.fori_loop(0, kn, eb, 0)

        @pl.when(nq > 0)
        def _():
            fire(0, 0)

        @pl.when(nq > 1)
        def _():
            fire(1, 1)

        def ring(t, carry):
            for j in range(NBUF):
                q = NBUF * t + j

                @pl.when(q < nq)
                def _(q=q, j=j):
                    @pl.when(q + 2 < nq)
                    def _():
                        fire(q + 2, (j + 2) % NBUF)
                    drain(j)
                    process(q, j)
            return carry
        lax.fori_loop(0, (nq + NBUF - 1) // NBUF, ring, 0)

        @pl.when(r == R - 1)
        def _():
            offo = pl.multiple_of(losub * HID, 8)
            pltpu.sync_copy(acc, out_h.at[pl.ds(offo, NPS * HID)])
        return carry
    lax.fori_loop(0, NP_, pp, 0)


def _sc_agg(table, hx, dl, wl, cu):
    mesh = plsc.VectorSubcoreMesh(core_axis_name="c", subcore_axis_name="s")
    k = functools.partial(
        pl.kernel,
        out_type=jax.ShapeDtypeStruct((NPAD * HID,), jnp.float32),
        mesh=mesh,
        compiler_params=pltpu.CompilerParams(needs_layout_passes=False),
        scratch_types=[
            pltpu.VMEM((NPS * HID,), jnp.float32),
            pltpu.VMEM((LSZ + L,), jnp.int32),
            pltpu.VMEM((LSZ + L,), jnp.int32),
            pltpu.VMEM((LSZ + L,), jnp.float32),
            pltpu.VMEM((2 * L,), jnp.int32),
            pltpu.VMEM((KCH,), jnp.int32),
            pltpu.VMEM((KCH,), jnp.int32),
            pltpu.VMEM((KCH,), jnp.int32),
            pltpu.VMEM((KCH, HID), jnp.float32),
            pltpu.VMEM((KCH, HID), jnp.float32),
            pltpu.VMEM((KCH, HID), jnp.float32),
            pltpu.SemaphoreType.DMA,
            pltpu.SemaphoreType.DMA,
            pltpu.SemaphoreType.DMA,
        ],
    )(_sc_layer_body)
    return k(table, hx, dl, wl, cu)


def _layer_agg(wh, hx, dl, wl, cu):
    table = wh.reshape(R * N, HID)
    return _sc_agg(table, hx, dl, wl, cu).reshape(NPAD, HID)[:N]


def kernel(x, edge_index, l1_baseW, l1_baseB, l1_dec_w, l1_dec_b, l1_statW,
           l1_statB, l2_baseW, l2_baseB, l2_dec_w, l2_dec_b, l2_statW,
           l2_statB):
    ei = edge_index.astype(jnp.int32)
    srcp = jnp.pad(ei[:, 0, :], ((0, 0), (0, EP - E))).reshape(-1)
    dstp = jnp.pad(ei[:, 1, :], ((0, 0), (0, EP - E)),
                   constant_values=PAD_DST).reshape(-1)

    hx, dl, wl, cu = _sc_prep(srcp, dstp)
    wh1, slf1 = _tc_layer((x,), _tc_l1_kernel, 768, l1_baseW, l1_baseB,
                          l1_dec_w, l1_dec_b, l1_statW, l1_statB)
    agg1 = _layer_agg(wh1, hx, dl, wl, cu)
    wh2, slf2 = _tc_layer((agg1, slf1), _tc_l2_kernel, HID, l2_baseW,
                          l2_baseB, l2_dec_w, l2_dec_b, l2_statW, l2_statB)
    agg2 = _layer_agg(wh2, hx, dl, wl, cu)
    return _tc_final(agg2, slf2)
